# baseline (speedup 1.0000x reference)
import numpy as np
import jax
import jax.numpy as jnp
from jax import lax
from jax.experimental import pallas as pl
from jax.experimental.pallas import tpu as pltpu

B = 16
L = 16
V = 100000
NROWS = B * L

TILE_S = 8
TILE_L = 512
TILE = TILE_S * TILE_L
NTILES = (V + TILE - 1) // TILE

BIG = np.int32(1 << 30)

_K1 = np.uint32(0)
_K2 = np.uint32(1234)
_K3 = np.uint32(0 ^ 1234 ^ 0x1BD11BDA)

_ONE = np.float32(1.0)
_TINY = np.float32(np.finfo(np.float32).tiny)
_SPAN = np.float32(_ONE - _TINY)
_CLIP = np.float32(1e-20)


def _rotl(x, d):
    return lax.shift_left(x, np.uint32(d)) | lax.shift_right_logical(
        x, np.uint32(32 - d))


def _four_rounds(x0, x1, rots):
    for r in rots:
        x0 = x0 + x1
        x1 = _rotl(x1, r) ^ x0
    return x0, x1


def _threefry_bits(count):
    x0 = jnp.zeros_like(count) + _K1
    x1 = count + _K2
    x0, x1 = _four_rounds(x0, x1, (13, 15, 26, 6))
    x0 = x0 + _K2
    x1 = x1 + np.uint32(_K3 + np.uint32(1))
    x0, x1 = _four_rounds(x0, x1, (17, 29, 16, 24))
    x0 = x0 + _K3
    x1 = x1 + np.uint32(_K1 + np.uint32(2))
    x0, x1 = _four_rounds(x0, x1, (13, 15, 26, 6))
    x0 = x0 + _K1
    x1 = x1 + np.uint32(_K2 + np.uint32(3))
    x0, x1 = _four_rounds(x0, x1, (17, 29, 16, 24))
    x0 = x0 + _K2
    x1 = x1 + np.uint32(_K3 + np.uint32(4))
    x0, x1 = _four_rounds(x0, x1, (13, 15, 26, 6))
    x0 = x0 + _K3
    x1 = x1 + np.uint32(_K1 + np.uint32(5))
    return x0 ^ x1


def _row_scan_kernel(ids_ref, red_ref):
    p = pl.program_id(0)
    idv = ids_ref[p // L, p % L]
    row_base = p * V

    iota_s = lax.broadcasted_iota(jnp.int32, (TILE_S, TILE_L), 0)
    iota_l = lax.broadcasted_iota(jnp.int32, (TILE_S, TILE_L), 1)
    v_base = iota_s * TILE_L + iota_l

    best_r = jnp.zeros((TILE_S, TILE_L), jnp.int32)
    best_i = jnp.full((TILE_S, TILE_L), BIG, jnp.int32)
    acc_rid = jnp.zeros((TILE_S, TILE_L), jnp.int32)
    r0_scalar = None

    for tidx in range(NTILES):
        v = v_base + (tidx * TILE)
        count = (v + row_base).astype(jnp.uint32)
        bits = _threefry_bits(count)
        r = lax.shift_right_logical(bits, np.uint32(9)).astype(jnp.int32)
        if tidx == NTILES - 1:
            r = jnp.where(v < V, r, 0)
        upd = r > best_r
        best_r = jnp.where(upd, r, best_r)
        best_i = jnp.where(upd, v, best_i)
        acc_rid = jnp.where(v == idv, r, acc_rid)
        if tidx == 0:
            r0_scalar = jnp.max(jnp.where(v == 0, r, 0))

    rmax = jnp.max(best_r)
    vmax = jnp.min(jnp.where(best_r == rmax, best_i, BIG))
    rid = jnp.max(acc_rid)

    red_ref[0, 0, 0] = vmax
    red_ref[0, 0, 1] = rmax
    red_ref[0, 0, 2] = r0_scalar
    red_ref[0, 0, 3] = rid


def _gumbel_from_r(r):
    fb = lax.bitcast_convert_type(r | jnp.int32(0x3F800000), jnp.float32)
    u = fb - _ONE
    up = jnp.maximum(_TINY, u * _SPAN + _TINY)
    return -jnp.log(-jnp.log(up))


def _merge_kernel(params_ref, ids_ref, t_ref, vmax_ref, rmax_ref, r0_ref,
                  rid_ref, out_ref):
    pi0 = params_ref[0]
    piu = params_ref[1]
    ids = ids_ref[...]
    trow = t_ref[...]
    vmax = vmax_ref[...]
    rmax = rmax_ref[...]
    r0 = r0_ref[...]
    rid = rid_ref[...]

    alpha = _ONE - trow
    is_mask = ids == 0
    pi_id = jnp.where(is_mask, pi0, piu)
    p_id = alpha + trow * pi_id
    c_id = jnp.log(jnp.maximum(p_id, _CLIP))
    p_0 = jnp.where(is_mask, p_id, trow * pi0)
    c_0 = jnp.log(jnp.maximum(p_0, _CLIP))
    c_oth = jnp.log(jnp.maximum(trow * piu, _CLIP))

    s_0 = c_0 + _gumbel_from_r(r0)
    s_id = c_id + _gumbel_from_r(rid)
    s_oth = c_oth + _gumbel_from_r(rmax)

    best_s = s_0
    best_v = jnp.zeros_like(ids)
    take = s_id > best_s
    best_s = jnp.where(take, s_id, best_s)
    best_v = jnp.where(take, ids, best_v)
    take = (s_oth > best_s) | ((s_oth == best_s) & (vmax < best_v))
    best_v = jnp.where(take, vmax, best_v)

    out_ref[...] = best_v


def _pad_rows(x, dtype):
    return jnp.pad(x.astype(dtype), (0, TILE_S * 128 - NROWS)).reshape(
        TILE_S, 128)


@jax.jit
def kernel(input_ids, t, pi):
    ids = input_ids.astype(jnp.int32)

    red = pl.pallas_call(
        _row_scan_kernel,
        grid=(NROWS,),
        in_specs=[pl.BlockSpec(memory_space=pltpu.SMEM)],
        out_specs=pl.BlockSpec((1, 1, 4), lambda p: (p, 0, 0),
                               memory_space=pltpu.SMEM),
        out_shape=jax.ShapeDtypeStruct((NROWS, 1, 4), jnp.int32),
        compiler_params=pltpu.CompilerParams(
            dimension_semantics=("parallel",)),
    )(ids)

    red = red[:, 0, :]
    params = jnp.stack([pi[0], pi[1]])
    ids_p = _pad_rows(ids.reshape(NROWS), jnp.int32)
    t_p = _pad_rows(jnp.repeat(t.astype(jnp.float32), L), jnp.float32)
    vmax_p = _pad_rows(red[:, 0], jnp.int32)
    rmax_p = _pad_rows(red[:, 1], jnp.int32)
    r0_p = _pad_rows(red[:, 2], jnp.int32)
    rid_p = _pad_rows(red[:, 3], jnp.int32)

    z = pl.pallas_call(
        _merge_kernel,
        in_specs=[pl.BlockSpec(memory_space=pltpu.SMEM)] +
                 [pl.BlockSpec(memory_space=pltpu.VMEM)] * 6,
        out_specs=pl.BlockSpec(memory_space=pltpu.VMEM),
        out_shape=jax.ShapeDtypeStruct((TILE_S, 128), jnp.int32),
    )(params, ids_p, t_p, vmax_p, rmax_p, r0_p, rid_p)

    return z.reshape(-1)[:NROWS].reshape(B, L)
